# baseline (speedup 1.0000x reference)
import jax
import jax.numpy as jnp
from jax import lax
from jax.experimental import pallas as pl

S = 2048
D = 768
R = 192
RP = 256
NI = 8
NP = 32
K = 3
NC = 3
TB = 256


def _body(x_ref, wr_ref, in_ref, pn_ref, out_ref):
    x = x_ref[...]
    scores = lax.dot_general(x, wr_ref[...], (((1,), (0,)), ((), ())),
                             preferred_element_type=jnp.float32)
    si = scores[:, :NI]
    sp = scores[:, NI:]
    si = si - jnp.max(si, axis=-1, keepdims=True)
    e = jnp.exp(si)
    w = e / jnp.sum(e, axis=-1, keepdims=True)

    xw = jnp.concatenate([x * w[:, n:n + 1] for n in range(NI)],
                         axis=1).astype(jnp.bfloat16)
    xr_all = lax.dot_general(xw, in_ref[...], (((1,), (0,)), ((), ())),
                             preferred_element_type=jnp.float32)
    xrs = [xr_all[:, c * RP:(c + 1) * RP] for c in range(NC)]

    pn = pn_ref[...]
    blocks = []
    for c in range(NC):
        blk = pn[:, c * RP:(c + 1) * RP]
        nrm = lax.rsqrt(jnp.sum(blk * blk, axis=-1, keepdims=True) + 1e-8)
        blocks.append(blk * nrm)
    pn_n = jnp.concatenate(blocks, axis=1)

    iota = lax.broadcasted_iota(jnp.int32, (TB, NP), 1)
    for _ in range(K):
        m = jnp.max(sp, axis=-1, keepdims=True)
        cand = jnp.where(sp == m, iota, NP)
        amin = jnp.min(cand, axis=-1, keepdims=True)
        oh = iota == amin
        sel = lax.dot_general(oh.astype(jnp.float32), pn_n,
                              (((1,), (0,)), ((), ())),
                              preferred_element_type=jnp.float32)
        for c in range(NC):
            sc = sel[:, c * RP:(c + 1) * RP]
            vtx = jnp.sum(xrs[c] * sc, axis=-1, keepdims=True)
            xrs[c] = xrs[c] - 2.0 * sc * vtx
        sp = jnp.where(oh, -jnp.inf, sp)

    for c in range(NC):
        out_ref[:, c * RP:c * RP + RP] = xrs[c]


def kernel(x, Wi, Wp, q_in, q_pn, k_in, k_pn, v_in, v_pn):
    x2 = x.reshape(S, D)
    wr = jnp.concatenate([Wi.T, Wp.T], axis=1)
    instk = jnp.stack([q_in, k_in, v_in])
    instk = jnp.pad(instk, ((0, 0), (0, 0), (0, 0), (0, RP - R)))
    instk = instk.transpose(1, 2, 0, 3).reshape(NI * D, NC * RP)
    instk = instk.astype(jnp.bfloat16)
    pnstk = jnp.stack([q_pn, k_pn, v_pn])
    pnstk = jnp.pad(pnstk, ((0, 0), (0, 0), (0, RP - R)))
    pnstk = pnstk.transpose(1, 0, 2).reshape(NP, NC * RP)
    out = pl.pallas_call(
        _body,
        grid=(S // TB,),
        in_specs=[
            pl.BlockSpec((TB, D), lambda t: (t, 0)),
            pl.BlockSpec((D, NI + NP), lambda t: (0, 0)),
            pl.BlockSpec((NI * D, NC * RP), lambda t: (0, 0)),
            pl.BlockSpec((NP, NC * RP), lambda t: (0, 0)),
        ],
        out_specs=pl.BlockSpec((TB, NC * RP), lambda t: (t, 0)),
        out_shape=jax.ShapeDtypeStruct((S, NC * RP), jnp.float32),
    )(x2, wr, instk, pnstk)
    return (out[:, 0:R].reshape(1, S, R),
            out[:, RP:RP + R].reshape(1, S, R),
            out[:, 2 * RP:2 * RP + R].reshape(1, S, R))


# speedup vs baseline: 1.6339x; 1.6339x over previous
import jax
import jax.numpy as jnp
from jax import lax
from jax.experimental import pallas as pl

S = 2048
D = 768
R = 192
RP = 256
NI = 8
NP = 32
K = 3
NC = 3
TB = 256


def _body(x_ref, wr_ref, in_ref, pn_ref, out_ref):
    x = x_ref[...]
    scores = lax.dot_general(x, wr_ref[...], (((1,), (0,)), ((), ())),
                             preferred_element_type=jnp.float32)
    si = scores[:, :NI]
    sp = scores[:, NI:]
    si = si - jnp.max(si, axis=-1, keepdims=True)
    e = jnp.exp(si)
    w = e / jnp.sum(e, axis=-1, keepdims=True)

    proj = lax.dot_general(x.astype(jnp.bfloat16), in_ref[...],
                           (((1,), (0,)), ((), ())),
                           preferred_element_type=jnp.float32)
    xrs = []
    for c in range(NC):
        xr = w[:, 0:1] * proj[:, c * NI * RP:c * NI * RP + RP]
        for n in range(1, NI):
            base = (c * NI + n) * RP
            xr = xr + w[:, n:n + 1] * proj[:, base:base + RP]
        xrs.append(xr)

    pn = pn_ref[...]
    blocks = []
    for c in range(NC):
        blk = pn[:, c * RP:(c + 1) * RP]
        nrm = lax.rsqrt(jnp.sum(blk * blk, axis=-1, keepdims=True) + 1e-8)
        blocks.append(blk * nrm)
    pn_n = jnp.concatenate(blocks, axis=1)

    iota = lax.broadcasted_iota(jnp.int32, (TB, NP), 1)
    for _ in range(K):
        m = jnp.max(sp, axis=-1, keepdims=True)
        cand = jnp.where(sp == m, iota, NP)
        amin = jnp.min(cand, axis=-1, keepdims=True)
        oh = iota == amin
        sel = lax.dot_general(oh.astype(jnp.float32), pn_n,
                              (((1,), (0,)), ((), ())),
                              preferred_element_type=jnp.float32)
        for c in range(NC):
            sc = sel[:, c * RP:(c + 1) * RP]
            vtx = jnp.sum(xrs[c] * sc, axis=-1, keepdims=True)
            xrs[c] = xrs[c] - 2.0 * sc * vtx
        sp = jnp.where(oh, -jnp.inf, sp)

    for c in range(NC):
        out_ref[:, c * RP:c * RP + RP] = xrs[c]


def kernel(x, Wi, Wp, q_in, q_pn, k_in, k_pn, v_in, v_pn):
    x2 = x.reshape(S, D)
    wr = jnp.concatenate([Wi.T, Wp.T], axis=1)
    instk = jnp.stack([q_in, k_in, v_in])
    instk = jnp.pad(instk, ((0, 0), (0, 0), (0, 0), (0, RP - R)))
    instk = instk.transpose(2, 0, 1, 3).reshape(D, NC * NI * RP)
    instk = instk.astype(jnp.bfloat16)
    pnstk = jnp.stack([q_pn, k_pn, v_pn])
    pnstk = jnp.pad(pnstk, ((0, 0), (0, 0), (0, RP - R)))
    pnstk = pnstk.transpose(1, 0, 2).reshape(NP, NC * RP)
    out = pl.pallas_call(
        _body,
        grid=(S // TB,),
        in_specs=[
            pl.BlockSpec((TB, D), lambda t: (t, 0)),
            pl.BlockSpec((D, NI + NP), lambda t: (0, 0)),
            pl.BlockSpec((D, NC * NI * RP), lambda t: (0, 0)),
            pl.BlockSpec((NP, NC * RP), lambda t: (0, 0)),
        ],
        out_specs=pl.BlockSpec((TB, NC * RP), lambda t: (t, 0)),
        out_shape=jax.ShapeDtypeStruct((S, NC * RP), jnp.float32),
    )(x2, wr, instk, pnstk)
    return (out[:, 0:R].reshape(1, S, R),
            out[:, RP:RP + R].reshape(1, S, R),
            out[:, 2 * RP:2 * RP + R].reshape(1, S, R))


# speedup vs baseline: 1.7274x; 1.0572x over previous
import jax
import jax.numpy as jnp
from jax import lax
from jax.experimental import pallas as pl

S = 2048
D = 768
R = 192
NI = 8
NP = 32
K = 3
NC = 3
TB = 256


def _body(x_ref, wr_ref, qin_ref, kin_ref, vin_ref, pn_ref,
          q_ref, k_ref, v_ref):
    x = x_ref[...]
    scores = lax.dot_general(x, wr_ref[...], (((1,), (0,)), ((), ())),
                             preferred_element_type=jnp.float32)
    si = scores[:, :NI]
    sp = scores[:, NI:]
    si = si - jnp.max(si, axis=-1, keepdims=True)
    e = jnp.exp(si)
    w = e / jnp.sum(e, axis=-1, keepdims=True)

    xrs = []
    for in_ref in (qin_ref, kin_ref, vin_ref):
        xr = w[:, 0:1] * lax.dot_general(
            x, in_ref[0], (((1,), (0,)), ((), ())),
            preferred_element_type=jnp.float32)
        for n in range(1, NI):
            xr = xr + w[:, n:n + 1] * lax.dot_general(
                x, in_ref[n], (((1,), (0,)), ((), ())),
                preferred_element_type=jnp.float32)
        xrs.append(xr)

    pn_ns = []
    for c in range(NC):
        blk = pn_ref[c]
        nrm = lax.rsqrt(jnp.sum(blk * blk, axis=-1, keepdims=True) + 1e-8)
        pn_ns.append(blk * nrm)

    iota = lax.broadcasted_iota(jnp.int32, (TB, NP), 1)
    for _ in range(K):
        m = jnp.max(sp, axis=-1, keepdims=True)
        cand = jnp.where(sp == m, iota, NP)
        amin = jnp.min(cand, axis=-1, keepdims=True)
        oh = (iota == amin)
        ohf = oh.astype(jnp.float32)
        for c in range(NC):
            sel = lax.dot_general(ohf, pn_ns[c], (((1,), (0,)), ((), ())),
                                  preferred_element_type=jnp.float32)
            vtx = jnp.sum(xrs[c] * sel, axis=-1, keepdims=True)
            xrs[c] = xrs[c] - 2.0 * sel * vtx
        sp = jnp.where(oh, -jnp.inf, sp)

    q_ref[...] = xrs[0]
    k_ref[...] = xrs[1]
    v_ref[...] = xrs[2]


def kernel(x, Wi, Wp, q_in, q_pn, k_in, k_pn, v_in, v_pn):
    x2 = x.reshape(S, D)
    wr = jnp.concatenate([Wi.T, Wp.T], axis=1)
    pnstk = jnp.stack([q_pn, k_pn, v_pn])
    full = lambda shape: pl.BlockSpec(shape, lambda t: tuple(0 for _ in shape))
    q, k, v = pl.pallas_call(
        _body,
        grid=(S // TB,),
        in_specs=[
            pl.BlockSpec((TB, D), lambda t: (t, 0)),
            full((D, NI + NP)),
            full((NI, D, R)),
            full((NI, D, R)),
            full((NI, D, R)),
            full((NC, NP, R)),
        ],
        out_specs=[
            pl.BlockSpec((TB, R), lambda t: (t, 0)),
            pl.BlockSpec((TB, R), lambda t: (t, 0)),
            pl.BlockSpec((TB, R), lambda t: (t, 0)),
        ],
        out_shape=[jax.ShapeDtypeStruct((S, R), jnp.float32)] * 3,
    )(x2, wr, q_in, k_in, v_in, pnstk)
    return (q.reshape(1, S, R), k.reshape(1, S, R), v.reshape(1, S, R))
